# baseline (speedup 1.0000x reference)
import functools

import jax
import jax.numpy as jnp
import numpy as np
from jax import lax
from jax.experimental import pallas as pl
from jax.experimental.pallas import tpu as pltpu
from jax.experimental.pallas import tpu_sc as plsc

NA, NB = 6000, 4000
FT, HID, HID2, OUT = 256, 256, 128, 64
K = 10
BR = 600


_SC_C = 4000
_SC_G = 64


def _make_seg_sum(E, n_dst, D):
    n_local = (-(-n_dst // 32) + 7) // 8 * 8
    n_pad = 32 * n_local
    nch = E // _SC_C
    grp = _SC_C // 16
    cnt_pad = ((n_local + 1 + 15) // 16) * 16
    mesh = plsc.VectorSubcoreMesh(core_axis_name="c", subcore_axis_name="s")

    @functools.partial(
        pl.kernel,
        out_type=(jax.ShapeDtypeStruct((n_pad, D), jnp.float32),
                  jax.ShapeDtypeStruct((n_pad, 16), jnp.float32)),
        mesh=mesh,
        compiler_params=pltpu.CompilerParams(needs_layout_passes=False),
        scratch_types=[
            pltpu.VMEM((_SC_C,), jnp.int32),
            pltpu.VMEM((_SC_C,), jnp.int32),
            pltpu.VMEM((_SC_C + _SC_G,), jnp.int32),
            pltpu.VMEM((_SC_C + _SC_G,), jnp.int32),
            pltpu.VMEM((_SC_G, D), jnp.float32),
            pltpu.VMEM((_SC_G, D), jnp.float32),
            pltpu.VMEM((n_local + 1, D), jnp.float32),
            pltpu.VMEM((n_local + 1, 16), jnp.float32),
            pltpu.SemaphoreType.DMA,
            pltpu.SemaphoreType.DMA,
        ],
    )
    def seg_sum(table, src, dst, out_sum, out_cnt, dstb, srcb, sel_s, sel_d,
                g0, g1, acc, cnt, sem0, sem1):
        w = lax.axis_index("s") * 2 + lax.axis_index("c")
        lo = w * n_local
        zf = jnp.zeros((16,), jnp.float32)

        def zacc(i, _):
            r = i // (D // 16)
            o = (i % (D // 16)) * 16
            acc[r, pl.ds(o, 16)] = zf
            return 0
        lax.fori_loop(0, (n_local + 1) * (D // 16), zacc, 0)

        def zcnt(i, _):
            cnt[i, :] = zf
            return 0
        lax.fori_loop(0, n_local + 1, zcnt, 0)
        e0 = jnp.where(lax.iota(jnp.int32, 16) == 0, 1.0, 0.0)

        def issue(j, gb, sem):
            pltpu.make_async_copy(
                table.at[sel_s.at[pl.ds(j * _SC_G, _SC_G)]], gb, sem).start()

        def waitb(j, gb, sem):
            pltpu.make_async_copy(
                table.at[sel_s.at[pl.ds(j * _SC_G, _SC_G)]], gb, sem).wait()

        def proc(gb, jj):
            base = jj * _SC_G

            def pgrp(g, _):
                dlv = sel_d[pl.ds(base + g * 16, 16)]
                for r in range(16):
                    dl = dlv[r]
                    gr = g * 16 + r
                    for kk in range(D // 16):
                        plsc.addupdate(acc.at[dl, pl.ds(kk * 16, 16)],
                                       gb[gr, pl.ds(kk * 16, 16)])
                    plsc.addupdate(cnt.at[dl, :], e0)
                return 0
            lax.fori_loop(0, _SC_G // 16, pgrp, 0)

        def chunk(ch, _):
            off = ch * _SC_C
            pltpu.sync_copy(dst.at[pl.ds(off, _SC_C)], dstb)
            pltpu.sync_copy(src.at[pl.ds(off, _SC_C)], srcb)

            def filt(i, nv):
                d = dstb[pl.ds(i * 16, 16)]
                sv = srcb[pl.ds(i * 16, 16)]
                dl = d - lo
                m = (dl >= 0) & (dl < n_local)
                pos = nv + plsc.cumsum(jnp.where(m, 1, 0)) - 1
                plsc.store_scatter(sel_s, [pos], sv, mask=m)
                plsc.store_scatter(sel_d, [pos], dl, mask=m)
                return nv + plsc.all_reduce_population_count(m)
            nv = lax.fori_loop(0, grp, filt, jnp.zeros((16,), jnp.int32))
            nsel = nv[0]

            zi = jnp.zeros((16,), jnp.int32)
            di = jnp.full((16,), n_local, jnp.int32)
            for t in range(_SC_G // 16):
                sel_s[pl.ds(nsel + t * 16, 16)] = zi
                sel_d[pl.ds(nsel + t * 16, 16)] = di
            nb = (nsel + _SC_G - 1) // _SC_G

            @pl.when(nb > 0)
            def _():
                issue(0, g0, sem0)

            def bpair(t, _):
                j0 = 2 * t
                j1 = j0 + 1

                @pl.when(j1 < nb)
                def _():
                    issue(j1, g1, sem1)
                waitb(j0, g0, sem0)
                proc(g0, j0)

                @pl.when(j0 + 2 < nb)
                def _():
                    issue(j0 + 2, g0, sem0)

                @pl.when(j1 < nb)
                def _():
                    waitb(j1, g1, sem1)
                    proc(g1, j1)
                return 0
            lax.fori_loop(0, (nb + 1) // 2, bpair, 0)
            return 0
        lax.fori_loop(0, nch, chunk, 0)

        pltpu.sync_copy(acc.at[pl.ds(0, n_local)], out_sum.at[pl.ds(lo, n_local)])
        pltpu.sync_copy(cnt.at[pl.ds(0, n_local)],
                        out_cnt.at[pl.ds(lo, n_local)])

    return seg_sum


_seg_sum_ba = _make_seg_sum(128000, NB, FT)
_seg_sum_ab = _make_seg_sum(192000, NA, HID)


def _sc_mean_agg(table, src, dst, n_dst, fn):
    s, c = fn(table, src.astype(jnp.int32), dst.astype(jnp.int32))
    return s[:n_dst] / jnp.maximum(c[:n_dst, 0], 1.0)[:, None]


def _a_assemble_body(idx_ref, w_ref, out_ref):
    cols = jax.lax.broadcasted_iota(jnp.int32, out_ref.shape, 1)
    acc = jnp.zeros(out_ref.shape, jnp.float32)
    for j in range(K):
        ij = idx_ref[:, j][:, None]
        wj = w_ref[:, j][:, None]
        acc = acc + jnp.where(ij == cols, wj, 0.0)
    out_ref[...] = acc


def _assemble_A(idxa0, w):
    return pl.pallas_call(
        _a_assemble_body,
        grid=(NA // BR,),
        in_specs=[
            pl.BlockSpec((BR, K), lambda i: (i, 0)),
            pl.BlockSpec((BR, K), lambda i: (i, 0)),
        ],
        out_specs=pl.BlockSpec((BR, NA), lambda i: (i, 0)),
        out_shape=jax.ShapeDtypeStruct((NA, NA), jnp.float32),
    )(idxa0, w)


def _mean_agg(feat_src, src, dst, n_dst):
    msg = jnp.take(feat_src, src, axis=0)
    s = jax.ops.segment_sum(msg, dst, num_segments=n_dst)
    cnt = jax.ops.segment_sum(jnp.ones((src.shape[0],), jnp.float32), dst,
                              num_segments=n_dst)
    return s / jnp.maximum(cnt, 1.0)[:, None]


def _spec_mlp(x, W0, b0, W1, b1):
    h = jax.nn.leaky_relu(x @ W0 + b0, negative_slope=0.01)
    return jnp.tanh(h @ W1 + b1)


def kernel(features, features_orth, edge_ab_src, edge_ab_dst, edge_ba_src,
           edge_ba_dst, idx, beta, alpha, W_bnn0_ab, W_bnn0_ba, W_bnn1_ab,
           W_bnn1_ba, W_fc_a, b_fc_a, W_fc_b, b_fc_b, W_sp0, b_sp0, W_sp1,
           b_sp1):
    feat_a = features[:NA]

    agg1 = _sc_mean_agg(features, edge_ba_src, edge_ba_dst, NB, _seg_sum_ba)
    embs1_b = jax.nn.relu(agg1 @ W_bnn0_ba)
    agg2 = _sc_mean_agg(embs1_b, edge_ab_src, edge_ab_dst, NA, _seg_sum_ab)
    v_a = jax.nn.relu(agg2 @ W_bnn1_ab)
    embs_het = v_a @ W_fc_a[:HID2] + feat_a @ W_fc_a[HID2:] + b_fc_a

    Yo = _spec_mlp(features_orth[:NA], W_sp0, b_sp0, W_sp1, b_sp1)
    _, R = jnp.linalg.qr(Yo)
    ow = np.sqrt(NA + 1e-08) * jnp.linalg.inv(R)
    Yt = _spec_mlp(features[:NA], W_sp0, b_sp0, W_sp1, b_sp1)
    Y = Yt @ ow
    Y_2 = Yt

    idxa0 = idx[:, 1:K + 1]
    dfi = jnp.sqrt(jnp.sum((Y[:, None, :] - Y[idxa0]) ** 2, axis=2) + 1e-08)
    ad = -(1.0 + beta[0]) * dfi / (2.0 * alpha[0])

    u = -jnp.sort(-ad, axis=1)
    css = jnp.cumsum(u, axis=1)
    ind = jnp.arange(1, K + 1, dtype=ad.dtype)
    cond = u * ind > (css - 1.0)
    rho = jnp.sum(cond, axis=1).astype(jnp.int32)
    theta = (jnp.take_along_axis(css, (rho - 1)[:, None], axis=1) - 1.0) \
        / rho[:, None].astype(ad.dtype)
    P = jnp.maximum(ad - theta, 0.0)

    eq = idxa0[:, :, None] == idxa0[:, None, :]
    later = jnp.triu(jnp.ones((K, K), bool), k=1)[None]
    dup_later = jnp.any(eq & later, axis=2)
    w = jnp.where(dup_later, 0.0, P)

    A = _assemble_A(idxa0, w)
    embs_hom = jnp.einsum("nk,nkd->nd", w, Y_2[idxa0])
    return (embs_het, embs_hom, A, Y)


# speedup vs baseline: 1.4304x; 1.4304x over previous
import functools

import jax
import jax.numpy as jnp
import numpy as np
from jax import lax
from jax.experimental import pallas as pl
from jax.experimental.pallas import tpu as pltpu
from jax.experimental.pallas import tpu_sc as plsc

NA, NB = 6000, 4000
FT, HID, HID2, OUT = 256, 256, 128, 64
K = 10
BR = 600


_SC_C = 1600
_SC_G = 32


def _make_seg_sum(E, n_dst, D, n_rows):
    n_local = (-(-n_dst // 32) + 7) // 8 * 8
    n_pad = 32 * n_local
    nch = E // _SC_C
    grp = _SC_C // 16
    cnt_pad = ((n_local + 1 + 15) // 16) * 16
    mesh = plsc.VectorSubcoreMesh(core_axis_name="c", subcore_axis_name="s")

    n_half = n_rows // 2

    @functools.partial(
        pl.kernel,
        out_type=(jax.ShapeDtypeStruct((2, n_pad, D), jnp.float32),
                  jax.ShapeDtypeStruct((2, n_pad, 16), jnp.float32)),
        mesh=mesh,
        compiler_params=pltpu.CompilerParams(needs_layout_passes=False),
        scratch_types=[
            pltpu.VMEM((_SC_C,), jnp.int32),
            pltpu.VMEM((_SC_C,), jnp.int32),
            pltpu.VMEM((_SC_C + _SC_G,), jnp.int32),
            pltpu.VMEM((_SC_C + _SC_G,), jnp.int32),
            pltpu.VMEM((_SC_G, D), jnp.float32),
            pltpu.VMEM((_SC_G, D), jnp.float32),
            pltpu.VMEM((n_local + 1, D), jnp.float32),
            pltpu.VMEM((n_local + 1, 16), jnp.float32),
            pltpu.VMEM_SHARED((n_half, D), jnp.float32),
            pltpu.SemaphoreType.DMA,
            pltpu.SemaphoreType.DMA,
        ],
    )
    def seg_sum(table, src, dst, out_sum, out_cnt, dstb, srcb, sel_s, sel_d,
                g0, g1, acc, cnt, spm, sem0, sem1):
        cc = lax.axis_index("c")
        w = lax.axis_index("s") * 2 + cc
        lo = w * n_local
        chalf = cc * n_half
        zf = jnp.zeros((16,), jnp.float32)

        nchk = n_half // 40
        rounds = -(-nchk // 16)
        sidx = lax.axis_index("s")

        def ld(r, _):
            ci = r * 16 + sidx

            @pl.when(ci < nchk)
            def _():
                pltpu.sync_copy(table.at[pl.ds(chalf + ci * 40, 40)],
                                g0.at[pl.ds(0, 40)])
                pltpu.sync_copy(g0.at[pl.ds(0, 40)],
                                spm.at[pl.ds(ci * 40, 40)])
            return 0
        lax.fori_loop(0, rounds, ld, 0)
        plsc.subcore_barrier()

        def zacc(i, _):
            r = i // (D // 16)
            o = (i % (D // 16)) * 16
            acc[r, pl.ds(o, 16)] = zf
            return 0
        lax.fori_loop(0, (n_local + 1) * (D // 16), zacc, 0)

        def zcnt(i, _):
            cnt[i, :] = zf
            return 0
        lax.fori_loop(0, n_local + 1, zcnt, 0)
        e0 = jnp.where(lax.iota(jnp.int32, 16) == 0, 1.0, 0.0)

        def issue(j, gb, sem):
            pltpu.make_async_copy(
                spm.at[sel_s.at[pl.ds(j * _SC_G, _SC_G)]], gb, sem).start()

        def waitb(j, gb, sem):
            pltpu.make_async_copy(
                spm.at[sel_s.at[pl.ds(j * _SC_G, _SC_G)]], gb, sem).wait()

        def proc(gb, jj):
            base = jj * _SC_G

            def pgrp(g, _):
                dlv = sel_d[pl.ds(base + g * 16, 16)]
                for r in range(16):
                    dl = dlv[r]
                    gr = g * 16 + r
                    for kk in range(D // 16):
                        plsc.addupdate(acc.at[dl, pl.ds(kk * 16, 16)],
                                       gb[gr, pl.ds(kk * 16, 16)])
                    plsc.addupdate(cnt.at[dl, :], e0)
                return 0
            lax.fori_loop(0, _SC_G // 16, pgrp, 0)

        def chunk(ch, _):
            off = ch * _SC_C
            pltpu.sync_copy(dst.at[pl.ds(off, _SC_C)], dstb)
            pltpu.sync_copy(src.at[pl.ds(off, _SC_C)], srcb)

            def filt(i, nv):
                d = dstb[pl.ds(i * 16, 16)]
                sv = srcb[pl.ds(i * 16, 16)]
                dl = d - lo
                sl = sv - chalf
                m = ((dl >= 0) & (dl < n_local)
                     & (sl >= 0) & (sl < n_half))
                pos = nv + plsc.cumsum(jnp.where(m, 1, 0)) - 1
                plsc.store_scatter(sel_s, [pos], sl, mask=m)
                plsc.store_scatter(sel_d, [pos], dl, mask=m)
                return nv + plsc.all_reduce_population_count(m)
            nv = lax.fori_loop(0, grp, filt, jnp.zeros((16,), jnp.int32))
            nsel = nv[0]

            zi = jnp.zeros((16,), jnp.int32)
            di = jnp.full((16,), n_local, jnp.int32)
            for t in range(_SC_G // 16):
                sel_s[pl.ds(nsel + t * 16, 16)] = zi
                sel_d[pl.ds(nsel + t * 16, 16)] = di
            nb = (nsel + _SC_G - 1) // _SC_G

            @pl.when(nb > 0)
            def _():
                issue(0, g0, sem0)

            def bpair(t, _):
                j0 = 2 * t
                j1 = j0 + 1

                @pl.when(j1 < nb)
                def _():
                    issue(j1, g1, sem1)
                waitb(j0, g0, sem0)
                proc(g0, j0)

                @pl.when(j0 + 2 < nb)
                def _():
                    issue(j0 + 2, g0, sem0)

                @pl.when(j1 < nb)
                def _():
                    waitb(j1, g1, sem1)
                    proc(g1, j1)
                return 0
            lax.fori_loop(0, (nb + 1) // 2, bpair, 0)
            return 0
        lax.fori_loop(0, nch, chunk, 0)

        pltpu.sync_copy(acc.at[pl.ds(0, n_local)],
                        out_sum.at[cc, pl.ds(lo, n_local)])
        pltpu.sync_copy(cnt.at[pl.ds(0, n_local)],
                        out_cnt.at[cc, pl.ds(lo, n_local)])

    return seg_sum


_seg_sum_ba = _make_seg_sum(128000, NB, FT, NA)
_seg_sum_ab = _make_seg_sum(192000, NA, HID, NB)


def _sc_mean_agg(table, src, dst, n_dst, fn):
    s, c = fn(table, src.astype(jnp.int32), dst.astype(jnp.int32))
    ssum = s[0, :n_dst] + s[1, :n_dst]
    csum = c[0, :n_dst, 0] + c[1, :n_dst, 0]
    return ssum / jnp.maximum(csum, 1.0)[:, None]


def _a_assemble_body(idx_ref, w_ref, out_ref):
    cols = jax.lax.broadcasted_iota(jnp.int32, out_ref.shape, 1)
    acc = jnp.zeros(out_ref.shape, jnp.float32)
    for j in range(K):
        ij = idx_ref[:, j][:, None]
        wj = w_ref[:, j][:, None]
        acc = acc + jnp.where(ij == cols, wj, 0.0)
    out_ref[...] = acc


def _assemble_A(idxa0, w):
    return pl.pallas_call(
        _a_assemble_body,
        grid=(NA // BR,),
        in_specs=[
            pl.BlockSpec((BR, K), lambda i: (i, 0)),
            pl.BlockSpec((BR, K), lambda i: (i, 0)),
        ],
        out_specs=pl.BlockSpec((BR, NA), lambda i: (i, 0)),
        out_shape=jax.ShapeDtypeStruct((NA, NA), jnp.float32),
    )(idxa0, w)


def _mean_agg_fold(feat_src, src, dst, n_dst):
    t1 = jnp.concatenate(
        [feat_src, jnp.ones((feat_src.shape[0], 1), jnp.float32)], axis=1)
    msg = jnp.take(t1, src, axis=0)
    s = jax.ops.segment_sum(msg, dst, num_segments=n_dst)
    return s[:, :-1] / jnp.maximum(s[:, -1], 1.0)[:, None]


def _spec_mlp(x, W0, b0, W1, b1):
    h = jax.nn.leaky_relu(x @ W0 + b0, negative_slope=0.01)
    return jnp.tanh(h @ W1 + b1)


def kernel(features, features_orth, edge_ab_src, edge_ab_dst, edge_ba_src,
           edge_ba_dst, idx, beta, alpha, W_bnn0_ab, W_bnn0_ba, W_bnn1_ab,
           W_bnn1_ba, W_fc_a, b_fc_a, W_fc_b, b_fc_b, W_sp0, b_sp0, W_sp1,
           b_sp1):
    feat_a = features[:NA]

    agg1 = _mean_agg_fold(feat_a, edge_ba_src, edge_ba_dst, NB)
    embs1_b = jax.nn.relu(agg1 @ W_bnn0_ba)
    agg2 = _mean_agg_fold(embs1_b, edge_ab_src, edge_ab_dst, NA)
    v_a = jax.nn.relu(agg2 @ W_bnn1_ab)
    embs_het = v_a @ W_fc_a[:HID2] + feat_a @ W_fc_a[HID2:] + b_fc_a

    Yo = _spec_mlp(features_orth[:NA], W_sp0, b_sp0, W_sp1, b_sp1)
    _, R = jnp.linalg.qr(Yo)
    ow = np.sqrt(NA + 1e-08) * jnp.linalg.inv(R)
    Yt = _spec_mlp(features[:NA], W_sp0, b_sp0, W_sp1, b_sp1)
    Y = Yt @ ow
    Y_2 = Yt

    idxa0 = idx[:, 1:K + 1]
    dfi = jnp.sqrt(jnp.sum((Y[:, None, :] - Y[idxa0]) ** 2, axis=2) + 1e-08)
    ad = -(1.0 + beta[0]) * dfi / (2.0 * alpha[0])

    u = -jnp.sort(-ad, axis=1)
    css = jnp.cumsum(u, axis=1)
    ind = jnp.arange(1, K + 1, dtype=ad.dtype)
    cond = u * ind > (css - 1.0)
    rho = jnp.sum(cond, axis=1).astype(jnp.int32)
    theta = (jnp.take_along_axis(css, (rho - 1)[:, None], axis=1) - 1.0) \
        / rho[:, None].astype(ad.dtype)
    P = jnp.maximum(ad - theta, 0.0)

    eq = idxa0[:, :, None] == idxa0[:, None, :]
    later = jnp.triu(jnp.ones((K, K), bool), k=1)[None]
    dup_later = jnp.any(eq & later, axis=2)
    w = jnp.where(dup_later, 0.0, P)

    A = _assemble_A(idxa0, w)
    embs_hom = jnp.einsum("nk,nkd->nd", w, Y_2[idxa0])
    return (embs_het, embs_hom, A, Y)


# speedup vs baseline: 1.4421x; 1.0082x over previous
import functools

import jax
import jax.numpy as jnp
import numpy as np
from jax import lax
from jax.experimental import pallas as pl
from jax.experimental.pallas import tpu as pltpu
from jax.experimental.pallas import tpu_sc as plsc

NA, NB = 6000, 4000
FT, HID, HID2, OUT = 256, 256, 128, 64
SPH = 512
K = 10
BR = 600


def _a_assemble_body(idx_ref, w_ref, out_ref):
    cols = jax.lax.broadcasted_iota(jnp.int32, out_ref.shape, 1)
    acc = jnp.zeros(out_ref.shape, jnp.float32)
    for j in range(K):
        ij = idx_ref[:, j][:, None]
        wj = w_ref[:, j][:, None]
        acc = acc + jnp.where(ij == cols, wj, 0.0)
    out_ref[...] = acc


def _assemble_A(idxa0, w):
    return pl.pallas_call(
        _a_assemble_body,
        grid=(NA // BR,),
        in_specs=[
            pl.BlockSpec((BR, K), lambda i: (i, 0)),
            pl.BlockSpec((BR, K), lambda i: (i, 0)),
        ],
        out_specs=pl.BlockSpec((BR, NA), lambda i: (i, 0)),
        out_shape=jax.ShapeDtypeStruct((NA, NA), jnp.float32),
    )(idxa0, w)


def _mean_agg_fold(feat_src, src, dst, n_dst):
    t1 = jnp.concatenate(
        [feat_src, jnp.ones((feat_src.shape[0], 1), jnp.float32)], axis=1)
    msg = jnp.take(t1, src, axis=0)
    s = jax.ops.segment_sum(msg, dst, num_segments=n_dst)
    return s[:, :-1] / jnp.maximum(s[:, -1], 1.0)[:, None]


def _mlp_body(x_ref, w0_ref, b0_ref, w1_ref, b1_ref, y_ref):
    h = jnp.dot(x_ref[...], w0_ref[...], preferred_element_type=jnp.float32)
    h = h + b0_ref[...]
    h = jnp.where(h >= 0.0, h, 0.01 * h)
    y = jnp.dot(h, w1_ref[...], preferred_element_type=jnp.float32)
    y_ref[...] = jnp.tanh(y + b1_ref[...])


def _spec_mlp(x, W0, b0, W1, b1):
    n = x.shape[0]
    blk = 1000
    return pl.pallas_call(
        _mlp_body,
        grid=(n // blk,),
        in_specs=[
            pl.BlockSpec((blk, FT), lambda i: (i, 0)),
            pl.BlockSpec((FT, SPH), lambda i: (0, 0)),
            pl.BlockSpec((1, SPH), lambda i: (0, 0)),
            pl.BlockSpec((SPH, OUT), lambda i: (0, 0)),
            pl.BlockSpec((1, OUT), lambda i: (0, 0)),
        ],
        out_specs=pl.BlockSpec((blk, OUT), lambda i: (i, 0)),
        out_shape=jax.ShapeDtypeStruct((n, OUT), jnp.float32),
    )(x, W0, b0.reshape(1, -1), W1, b1.reshape(1, -1))


def _qr_ow_body(yo_ref, ow_ref, mt_scr, x_scr, r_scr):
    n = yo_ref.shape[0]
    mt_scr[...] = yo_ref[...].T
    pos = jax.lax.broadcasted_iota(jnp.int32, (1, n), 1)
    pos64 = jax.lax.broadcasted_iota(jnp.int32, (1, OUT), 1)

    def step(j, _):
        x = mt_scr[pl.ds(j, 1), :]
        alpha = jnp.sum(jnp.where(pos == j, x, 0.0))
        xm = jnp.where(pos >= j, x, 0.0)
        sigma = jnp.sqrt(jnp.sum(xm * xm))
        beta = jnp.where(alpha >= 0.0, -sigma, sigma)
        v = jnp.where(pos > j, xm, 0.0) + jnp.where(pos == j, alpha - beta,
                                                    0.0)
        vtv = jnp.sum(v * v)
        scale = jnp.where(vtv > 0.0, 2.0 / vtv, 0.0)
        w = jnp.dot(mt_scr[...], v.reshape(n, 1),
                    preferred_element_type=jnp.float32)
        mt_scr[...] = mt_scr[...] - (scale * w) * v
        return 0
    jax.lax.fori_loop(0, OUT, step, 0)

    r_scr[...] = mt_scr[:, :OUT].T
    x_scr[...] = jnp.zeros((OUT, OUT), jnp.float32)

    def back(t, _):
        i = OUT - 1 - t
        ri = r_scr[pl.ds(i, 1), :]
        rii = jnp.sum(jnp.where(pos64 == i, ri, 0.0))
        rup = jnp.where(pos64 > i, ri, 0.0)
        ei = jnp.where(pos64 == i, 1.0, 0.0)
        acc = jnp.dot(rup, x_scr[...], preferred_element_type=jnp.float32)
        x_scr[pl.ds(i, 1), :] = (ei - acc) / rii
        return 0
    jax.lax.fori_loop(0, OUT, back, 0)
    ow_ref[...] = np.sqrt(NA + 1e-08).astype(np.float32) * x_scr[...]


def _qr_ow(yo):
    n = yo.shape[0]
    return pl.pallas_call(
        _qr_ow_body,
        scratch_shapes=[
            pltpu.VMEM((OUT, n), jnp.float32),
            pltpu.VMEM((OUT, OUT), jnp.float32),
            pltpu.VMEM((OUT, OUT), jnp.float32),
        ],
        out_shape=jax.ShapeDtypeStruct((OUT, OUT), jnp.float32),
    )(yo)


def _aff_body(yt_ref, g2_ref, idx_ref, ow_ref, coef_ref, y_ref, w_ref,
              hom_ref):
    coef = coef_ref[0, 0]
    ow = ow_ref[...]
    y = jnp.dot(yt_ref[...], ow, preferred_element_type=jnp.float32)
    y_ref[...] = y
    ad = []
    for j in range(K):
        g2j = g2_ref[:, j * OUT:(j + 1) * OUT]
        ynj = jnp.dot(g2j, ow, preferred_element_type=jnp.float32)
        d = y - ynj
        dfi = jnp.sqrt(jnp.sum(d * d, axis=1, keepdims=True) + 1e-08)
        ad.append(coef * dfi)
    u = list(ad)
    for r in range(K):
        for p in range(r % 2, K - 1, 2):
            hi = jnp.maximum(u[p], u[p + 1])
            lo = jnp.minimum(u[p], u[p + 1])
            u[p], u[p + 1] = hi, lo
    css = []
    run = jnp.zeros_like(u[0])
    for j in range(K):
        run = run + u[j]
        css.append(run)
    rho = jnp.zeros_like(u[0])
    for j in range(K):
        rho = rho + jnp.where(u[j] * (j + 1.0) > css[j] - 1.0, 1.0, 0.0)
    theta_num = jnp.zeros_like(u[0])
    for j in range(K):
        theta_num = theta_num + jnp.where(rho == (j + 1.0), css[j], 0.0)
    theta = (theta_num - 1.0) / rho
    hom = jnp.zeros((yt_ref.shape[0], OUT), jnp.float32)
    for j in range(K):
        pj = jnp.maximum(ad[j] - theta, 0.0)
        dup = jnp.zeros_like(pj, dtype=jnp.bool_)
        ij = idx_ref[:, j][:, None]
        for j2 in range(j + 1, K):
            dup = dup | (ij == idx_ref[:, j2][:, None])
        wj = jnp.where(dup, 0.0, pj)
        w_ref[:, pl.ds(j, 1)] = wj
        hom = hom + wj * g2_ref[:, j * OUT:(j + 1) * OUT]
    hom_ref[...] = hom


def _affinity(Yt, G2flat, idxa0, ow, coef):
    blk = 600
    return pl.pallas_call(
        _aff_body,
        grid=(NA // blk,),
        in_specs=[
            pl.BlockSpec((blk, OUT), lambda i: (i, 0)),
            pl.BlockSpec((blk, K * OUT), lambda i: (i, 0)),
            pl.BlockSpec((blk, K), lambda i: (i, 0)),
            pl.BlockSpec((OUT, OUT), lambda i: (0, 0)),
            pl.BlockSpec(memory_space=pltpu.SMEM),
        ],
        out_specs=[
            pl.BlockSpec((blk, OUT), lambda i: (i, 0)),
            pl.BlockSpec((blk, K), lambda i: (i, 0)),
            pl.BlockSpec((blk, OUT), lambda i: (i, 0)),
        ],
        out_shape=[
            jax.ShapeDtypeStruct((NA, OUT), jnp.float32),
            jax.ShapeDtypeStruct((NA, K), jnp.float32),
            jax.ShapeDtypeStruct((NA, OUT), jnp.float32),
        ],
    )(Yt, G2flat, idxa0, ow, coef)


def kernel(features, features_orth, edge_ab_src, edge_ab_dst, edge_ba_src,
           edge_ba_dst, idx, beta, alpha, W_bnn0_ab, W_bnn0_ba, W_bnn1_ab,
           W_bnn1_ba, W_fc_a, b_fc_a, W_fc_b, b_fc_b, W_sp0, b_sp0, W_sp1,
           b_sp1):
    feat_a = features[:NA]

    agg1 = _mean_agg_fold(feat_a, edge_ba_src, edge_ba_dst, NB)
    embs1_b = jax.nn.relu(agg1 @ W_bnn0_ba)
    agg2 = _mean_agg_fold(embs1_b, edge_ab_src, edge_ab_dst, NA)
    v_a = jax.nn.relu(agg2 @ W_bnn1_ab)
    embs_het = v_a @ W_fc_a[:HID2] + feat_a @ W_fc_a[HID2:] + b_fc_a

    Yo = _spec_mlp(features_orth[:NA], W_sp0, b_sp0, W_sp1, b_sp1)
    ow = _qr_ow(Yo)
    Yt = _spec_mlp(features[:NA], W_sp0, b_sp0, W_sp1, b_sp1)

    idxa0 = idx[:, 1:K + 1].astype(jnp.int32)
    G2 = jnp.take(Yt, idxa0.reshape(-1), axis=0).reshape(NA, K * OUT)
    coef = (-(1.0 + beta[0]) / (2.0 * alpha[0])).reshape(1, 1)
    Y, w, embs_hom = _affinity(Yt, G2, idxa0, ow, coef)
    A = _assemble_A(idxa0, w)
    return (embs_het, embs_hom, A, Y)


# speedup vs baseline: 2.1354x; 1.4808x over previous
import functools

import jax
import jax.numpy as jnp
import numpy as np
from jax import lax
from jax.experimental import pallas as pl
from jax.experimental.pallas import tpu as pltpu
from jax.experimental.pallas import tpu_sc as plsc

NA, NB = 6000, 4000
FT, HID, HID2, OUT = 256, 256, 128, 64
SPH = 512
K = 10
BR = 600


_SC_C = 4000
_SC_G = 16
_SC_NB = 8


def _make_seg_sum(E, n_dst, D):
    n_local = (-(-n_dst // 32) + 7) // 8 * 8
    n_pad = 32 * n_local
    nch = E // _SC_C
    grp = _SC_C // 16
    mesh = plsc.VectorSubcoreMesh(core_axis_name="c", subcore_axis_name="s")

    @functools.partial(
        pl.kernel,
        out_type=(jax.ShapeDtypeStruct((n_pad, D), jnp.float32),
                  jax.ShapeDtypeStruct((n_pad, 16), jnp.float32)),
        mesh=mesh,
        compiler_params=pltpu.CompilerParams(needs_layout_passes=False),
        scratch_types=(
            [pltpu.VMEM((_SC_C,), jnp.int32),
             pltpu.VMEM((_SC_C,), jnp.int32),
             pltpu.VMEM((_SC_C + _SC_NB * _SC_G,), jnp.int32),
             pltpu.VMEM((_SC_C + _SC_NB * _SC_G,), jnp.int32)]
            + [pltpu.VMEM((_SC_G, D), jnp.float32)] * _SC_NB
            + [pltpu.VMEM((n_local + 1, D), jnp.float32),
               pltpu.VMEM((n_local + 1, 16), jnp.float32)]
            + [pltpu.SemaphoreType.DMA] * _SC_NB
        ),
    )
    def seg_sum(table, src, dst, out_sum, out_cnt, dstb, srcb, sel_s, sel_d,
                *rest):
        gbufs = rest[:_SC_NB]
        acc = rest[_SC_NB]
        cnt = rest[_SC_NB + 1]
        sems = rest[_SC_NB + 2:]
        w = lax.axis_index("s") * 2 + lax.axis_index("c")
        lo = w * n_local
        zf = jnp.zeros((16,), jnp.float32)

        def zacc(i, _):
            r = i // (D // 16)
            o = (i % (D // 16)) * 16
            acc[r, pl.ds(o, 16)] = zf
            return 0
        lax.fori_loop(0, (n_local + 1) * (D // 16), zacc, 0)

        def zcnt(i, _):
            cnt[i, :] = zf
            return 0
        lax.fori_loop(0, n_local + 1, zcnt, 0)
        e0 = jnp.where(lax.iota(jnp.int32, 16) == 0, 1.0, 0.0)

        def issue(j, b):
            pltpu.make_async_copy(
                table.at[sel_s.at[pl.ds(j * _SC_G, _SC_G)]],
                gbufs[b], sems[b]).start()

        def waitb(j, b):
            pltpu.make_async_copy(
                table.at[sel_s.at[pl.ds(j * _SC_G, _SC_G)]],
                gbufs[b], sems[b]).wait()

        def proc(j, b):
            gb = gbufs[b]
            dlv = sel_d[pl.ds(j * _SC_G, 16)]
            for r in range(16):
                dl = dlv[r]
                for kk in range(D // 16):
                    plsc.addupdate(acc.at[dl, pl.ds(kk * 16, 16)],
                                   gb[r, pl.ds(kk * 16, 16)])
                plsc.addupdate(cnt.at[dl, :], e0)

        def chunk(ch, _):
            off = ch * _SC_C
            pltpu.sync_copy(dst.at[pl.ds(off, _SC_C)], dstb)
            pltpu.sync_copy(src.at[pl.ds(off, _SC_C)], srcb)

            def filt(i, nv):
                d = dstb[pl.ds(i * 16, 16)]
                sv = srcb[pl.ds(i * 16, 16)]
                dl = d - lo
                m = (dl >= 0) & (dl < n_local)
                pos = nv + plsc.cumsum(jnp.where(m, 1, 0)) - 1
                plsc.store_scatter(sel_s, [pos], sv, mask=m)
                plsc.store_scatter(sel_d, [pos], dl, mask=m)
                return nv + plsc.all_reduce_population_count(m)
            nv = lax.fori_loop(0, grp, filt, jnp.zeros((16,), jnp.int32))
            nsel = nv[0]

            zi = jnp.zeros((16,), jnp.int32)
            di = jnp.full((16,), n_local, jnp.int32)
            for t in range(_SC_NB):
                sel_s[pl.ds(nsel + t * 16, 16)] = zi
                sel_d[pl.ds(nsel + t * 16, 16)] = di
            nb = (nsel + _SC_G - 1) // _SC_G

            def fire_group(t, _):
                j0 = t * _SC_NB
                for b in range(_SC_NB):
                    @pl.when(j0 + b < nb)
                    def _():
                        issue(j0 + b, b)
                for b in range(_SC_NB):
                    @pl.when(j0 + b < nb)
                    def _():
                        waitb(j0 + b, b)
                        proc(j0 + b, b)
                return 0
            lax.fori_loop(0, (nb + _SC_NB - 1) // _SC_NB, fire_group, 0)
            return 0
        lax.fori_loop(0, nch, chunk, 0)

        pltpu.sync_copy(acc.at[pl.ds(0, n_local)],
                        out_sum.at[pl.ds(lo, n_local)])
        pltpu.sync_copy(cnt.at[pl.ds(0, n_local)],
                        out_cnt.at[pl.ds(lo, n_local)])

    return seg_sum


_seg_sum_ba = _make_seg_sum(128000, NB, FT)
_seg_sum_ab = _make_seg_sum(192000, NA, HID)


def _sc_mean_agg(table, src, dst, n_dst, fn):
    s, c = fn(table, src.astype(jnp.int32), dst.astype(jnp.int32))
    return s[:n_dst] / jnp.maximum(c[:n_dst, 0], 1.0)[:, None]


def _a_assemble_body(idx_ref, w_ref, out_ref):
    cols = jax.lax.broadcasted_iota(jnp.int32, out_ref.shape, 1)
    acc = jnp.zeros(out_ref.shape, jnp.float32)
    for j in range(K):
        ij = idx_ref[:, j][:, None]
        wj = w_ref[:, j][:, None]
        acc = acc + jnp.where(ij == cols, wj, 0.0)
    out_ref[...] = acc


def _assemble_A(idxa0, w):
    return pl.pallas_call(
        _a_assemble_body,
        grid=(NA // BR,),
        in_specs=[
            pl.BlockSpec((BR, K), lambda i: (i, 0)),
            pl.BlockSpec((BR, K), lambda i: (i, 0)),
        ],
        out_specs=pl.BlockSpec((BR, NA), lambda i: (i, 0)),
        out_shape=jax.ShapeDtypeStruct((NA, NA), jnp.float32),
    )(idxa0, w)


def _mean_agg_fold(feat_src, src, dst, n_dst):
    t1 = jnp.concatenate(
        [feat_src, jnp.ones((feat_src.shape[0], 1), jnp.float32)], axis=1)
    msg = jnp.take(t1, src, axis=0)
    s = jax.ops.segment_sum(msg, dst, num_segments=n_dst)
    return s[:, :-1] / jnp.maximum(s[:, -1], 1.0)[:, None]


def _mlp_body(x_ref, w0_ref, b0_ref, w1_ref, b1_ref, y_ref):
    h = jnp.dot(x_ref[...], w0_ref[...], preferred_element_type=jnp.float32)
    h = h + b0_ref[...]
    h = jnp.where(h >= 0.0, h, 0.01 * h)
    y = jnp.dot(h, w1_ref[...], preferred_element_type=jnp.float32)
    y_ref[...] = jnp.tanh(y + b1_ref[...])


def _spec_mlp(x, W0, b0, W1, b1):
    n = x.shape[0]
    blk = 1000
    return pl.pallas_call(
        _mlp_body,
        grid=(n // blk,),
        in_specs=[
            pl.BlockSpec((blk, FT), lambda i: (i, 0)),
            pl.BlockSpec((FT, SPH), lambda i: (0, 0)),
            pl.BlockSpec((1, SPH), lambda i: (0, 0)),
            pl.BlockSpec((SPH, OUT), lambda i: (0, 0)),
            pl.BlockSpec((1, OUT), lambda i: (0, 0)),
        ],
        out_specs=pl.BlockSpec((blk, OUT), lambda i: (i, 0)),
        out_shape=jax.ShapeDtypeStruct((n, OUT), jnp.float32),
    )(x, W0, b0.reshape(1, -1), W1, b1.reshape(1, -1))


def _qr_ow_body(yo_ref, ow_ref, mt_scr, x_scr, r_scr):
    n = yo_ref.shape[0]
    mt_scr[...] = yo_ref[...].T
    pos = jax.lax.broadcasted_iota(jnp.int32, (1, n), 1)
    pos64 = jax.lax.broadcasted_iota(jnp.int32, (1, OUT), 1)

    def step(j, _):
        x = mt_scr[pl.ds(j, 1), :]
        alpha = jnp.sum(jnp.where(pos == j, x, 0.0))
        xm = jnp.where(pos >= j, x, 0.0)
        sigma = jnp.sqrt(jnp.sum(xm * xm))
        beta = jnp.where(alpha >= 0.0, -sigma, sigma)
        v = jnp.where(pos > j, xm, 0.0) + jnp.where(pos == j, alpha - beta,
                                                    0.0)
        vtv = jnp.sum(v * v)
        scale = jnp.where(vtv > 0.0, 2.0 / vtv, 0.0)
        w = jnp.dot(mt_scr[...], v.reshape(n, 1),
                    preferred_element_type=jnp.float32)
        mt_scr[...] = mt_scr[...] - (scale * w) * v
        return 0
    jax.lax.fori_loop(0, OUT, step, 0)

    r_scr[...] = mt_scr[:, :OUT].T
    x_scr[...] = jnp.zeros((OUT, OUT), jnp.float32)

    def back(t, _):
        i = OUT - 1 - t
        ri = r_scr[pl.ds(i, 1), :]
        rii = jnp.sum(jnp.where(pos64 == i, ri, 0.0))
        rup = jnp.where(pos64 > i, ri, 0.0)
        ei = jnp.where(pos64 == i, 1.0, 0.0)
        acc = jnp.dot(rup, x_scr[...], preferred_element_type=jnp.float32)
        x_scr[pl.ds(i, 1), :] = (ei - acc) / rii
        return 0
    jax.lax.fori_loop(0, OUT, back, 0)
    ow_ref[...] = np.sqrt(NA + 1e-08).astype(np.float32) * x_scr[...]


def _qr_ow(yo):
    n = yo.shape[0]
    return pl.pallas_call(
        _qr_ow_body,
        scratch_shapes=[
            pltpu.VMEM((OUT, n), jnp.float32),
            pltpu.VMEM((OUT, OUT), jnp.float32),
            pltpu.VMEM((OUT, OUT), jnp.float32),
        ],
        out_shape=jax.ShapeDtypeStruct((OUT, OUT), jnp.float32),
    )(yo)


def _aff_body(yt_ref, g2_ref, idx_ref, ow_ref, coef_ref, y_ref, w_ref,
              hom_ref):
    coef = coef_ref[0, 0]
    ow = ow_ref[...]
    y = jnp.dot(yt_ref[...], ow, preferred_element_type=jnp.float32)
    y_ref[...] = y
    ad = []
    for j in range(K):
        g2j = g2_ref[:, j * OUT:(j + 1) * OUT]
        ynj = jnp.dot(g2j, ow, preferred_element_type=jnp.float32)
        d = y - ynj
        dfi = jnp.sqrt(jnp.sum(d * d, axis=1, keepdims=True) + 1e-08)
        ad.append(coef * dfi)
    u = list(ad)
    for r in range(K):
        for p in range(r % 2, K - 1, 2):
            hi = jnp.maximum(u[p], u[p + 1])
            lo = jnp.minimum(u[p], u[p + 1])
            u[p], u[p + 1] = hi, lo
    css = []
    run = jnp.zeros_like(u[0])
    for j in range(K):
        run = run + u[j]
        css.append(run)
    rho = jnp.zeros_like(u[0])
    for j in range(K):
        rho = rho + jnp.where(u[j] * (j + 1.0) > css[j] - 1.0, 1.0, 0.0)
    theta_num = jnp.zeros_like(u[0])
    for j in range(K):
        theta_num = theta_num + jnp.where(rho == (j + 1.0), css[j], 0.0)
    theta = (theta_num - 1.0) / rho
    hom = jnp.zeros((yt_ref.shape[0], OUT), jnp.float32)
    for j in range(K):
        pj = jnp.maximum(ad[j] - theta, 0.0)
        dup = jnp.zeros_like(pj, dtype=jnp.bool_)
        ij = idx_ref[:, j][:, None]
        for j2 in range(j + 1, K):
            dup = dup | (ij == idx_ref[:, j2][:, None])
        wj = jnp.where(dup, 0.0, pj)
        w_ref[:, pl.ds(j, 1)] = wj
        hom = hom + wj * g2_ref[:, j * OUT:(j + 1) * OUT]
    hom_ref[...] = hom


def _affinity(Yt, G2flat, idxa0, ow, coef):
    blk = 600
    return pl.pallas_call(
        _aff_body,
        grid=(NA // blk,),
        in_specs=[
            pl.BlockSpec((blk, OUT), lambda i: (i, 0)),
            pl.BlockSpec((blk, K * OUT), lambda i: (i, 0)),
            pl.BlockSpec((blk, K), lambda i: (i, 0)),
            pl.BlockSpec((OUT, OUT), lambda i: (0, 0)),
            pl.BlockSpec(memory_space=pltpu.SMEM),
        ],
        out_specs=[
            pl.BlockSpec((blk, OUT), lambda i: (i, 0)),
            pl.BlockSpec((blk, K), lambda i: (i, 0)),
            pl.BlockSpec((blk, OUT), lambda i: (i, 0)),
        ],
        out_shape=[
            jax.ShapeDtypeStruct((NA, OUT), jnp.float32),
            jax.ShapeDtypeStruct((NA, K), jnp.float32),
            jax.ShapeDtypeStruct((NA, OUT), jnp.float32),
        ],
    )(Yt, G2flat, idxa0, ow, coef)


def kernel(features, features_orth, edge_ab_src, edge_ab_dst, edge_ba_src,
           edge_ba_dst, idx, beta, alpha, W_bnn0_ab, W_bnn0_ba, W_bnn1_ab,
           W_bnn1_ba, W_fc_a, b_fc_a, W_fc_b, b_fc_b, W_sp0, b_sp0, W_sp1,
           b_sp1):
    feat_a = features[:NA]

    agg1 = _sc_mean_agg(feat_a, edge_ba_src, edge_ba_dst, NB, _seg_sum_ba)
    embs1_b = jax.nn.relu(agg1 @ W_bnn0_ba)
    agg2 = _sc_mean_agg(embs1_b, edge_ab_src, edge_ab_dst, NA, _seg_sum_ab)
    v_a = jax.nn.relu(agg2 @ W_bnn1_ab)
    embs_het = v_a @ W_fc_a[:HID2] + feat_a @ W_fc_a[HID2:] + b_fc_a

    Yo = _spec_mlp(features_orth[:NA], W_sp0, b_sp0, W_sp1, b_sp1)
    ow = _qr_ow(Yo)
    Yt = _spec_mlp(features[:NA], W_sp0, b_sp0, W_sp1, b_sp1)

    idxa0 = idx[:, 1:K + 1].astype(jnp.int32)
    G2 = jnp.take(Yt, idxa0.reshape(-1), axis=0).reshape(NA, K * OUT)
    coef = (-(1.0 + beta[0]) / (2.0 * alpha[0])).reshape(1, 1)
    Y, w, embs_hom = _affinity(Yt, G2, idxa0, ow, coef)
    A = _assemble_A(idxa0, w)
    return (embs_het, embs_hom, A, Y)
